# baseline (speedup 1.0000x reference)
import functools

import jax
import jax.numpy as jnp
from jax import lax
from jax.experimental import pallas as pl
from jax.experimental.pallas import tpu as pltpu
from jax.experimental.pallas import tpu_sc as plsc

N = 10000
E = 320000
D = 128
G = 64

NC = 2
NS = 16
DH = D // NC
ET = E // NS
K = 128
CPB = 8
NBLK = 20
ETP = NBLK * CPB * K
PADT = ETP - ET
NP = 10112
RPT = NP // NS
NPAD = NP - N


def _sc_body(esrc_hbm, edst_hbm, xs_hbm, out_hbm, sidx, didx,
             b0, b1, b2, b3, g0, g1, g2, g3, s0, s1, s2, s3, isem,
             shared):
    c = lax.axis_index("c")
    s = lax.axis_index("s")
    bufs = (b0, b1, b2, b3)
    gsems = (g0, g1, g2, g3)
    ssems = (s0, s1, s2, s3)

    zero = jnp.zeros((16,), jnp.float32)

    def _zfill(i, carry):
        b0[i // 4, pl.ds((i % 4) * 16, 16)] = zero
        return carry

    lax.fori_loop(0, K * 4, _zfill, 0)
    for z in range(4):
        pltpu.sync_copy(b0, shared.at[pl.ds(s * RPT + z * K, K)])
    pltpu.sync_copy(b0.at[pl.ds(0, RPT - 4 * K)],
                    shared.at[pl.ds(s * RPT + 4 * K, RPT - 4 * K)])

    pltpu.async_copy(xs_hbm.at[pl.ds(0, K)], b2, s2)
    pltpu.async_copy(xs_hbm.at[pl.ds(K, K)], b3, s3)

    plsc.subcore_barrier()

    def _gather(ph, j, q):
        pltpu.async_copy(xs_hbm.at[sidx.at[ph, j]], bufs[q], gsems[q])

    def _gwait(ph, j, q):
        pltpu.make_async_copy(xs_hbm.at[sidx.at[ph, j]], bufs[q],
                              gsems[q]).wait()

    def _scatter(ph, j, q):
        pltpu.async_copy(bufs[q], shared.at[didx.at[ph, j]], ssems[q],
                         add=True)

    def _swait(q):
        pltpu.make_async_copy(bufs[q], shared.at[didx.at[0, 0]],
                              ssems[q]).wait()

    pltpu.sync_copy(esrc_hbm.at[c, s, 0], sidx.at[0])
    pltpu.sync_copy(edst_hbm.at[s, 0], didx.at[0])
    _gather(0, 0, 0)
    _gather(0, 1, 1)

    def _block(blk, carry):
        ph = lax.rem(blk, 2)
        for j in range(CPB):
            tgt = j + 2
            if tgt < CPB:
                _swait(tgt % 4)
                _gather(ph, tgt, tgt % 4)
            else:
                if j == CPB - 2:
                    @pl.when(blk < NBLK - 1)
                    def _():
                        pltpu.make_async_copy(
                            esrc_hbm.at[c, s, blk + 1], sidx.at[1 - ph],
                            isem).wait()
                        pltpu.make_async_copy(
                            edst_hbm.at[s, blk + 1], didx.at[1 - ph],
                            isem).wait()

                @pl.when(blk < NBLK - 1)
                def _():
                    _swait(tgt % 4)
                    _gather(1 - ph, tgt - CPB, tgt % 4)
            if j == 2:
                @pl.when(blk < NBLK - 1)
                def _():
                    pltpu.async_copy(esrc_hbm.at[c, s, blk + 1],
                                     sidx.at[1 - ph], isem)
                    pltpu.async_copy(edst_hbm.at[s, blk + 1],
                                     didx.at[1 - ph], isem)
            _gwait(ph, j, j % 4)
            _scatter(ph, j, j % 4)
        return carry

    lax.fori_loop(0, NBLK, _block, 0)
    for q in range(4):
        _swait(q)

    plsc.subcore_barrier()
    pltpu.sync_copy(shared.at[pl.ds(s * RPT, RPT)],
                    out_hbm.at[c, pl.ds(s * RPT, RPT)])


@jax.jit
def _sc_aggregate(edge_index, x):
    xs = jnp.concatenate([x[:, :DH], x[:, DH:]], axis=0)
    t = jnp.arange(NS, dtype=jnp.int32)[:, None]
    p = jnp.arange(PADT, dtype=jnp.int32)
    pad_src = (t * 317 + p * 13) % N
    pad_dst = N + (t * 31 + p) % NPAD
    src = jnp.concatenate([edge_index[0].reshape(NS, ET), pad_src], axis=1)
    dst = jnp.concatenate([edge_index[1].reshape(NS, ET), pad_dst], axis=1)
    coff = jnp.arange(NC, dtype=jnp.int32)[:, None, None] * N
    esrc = (src[None] + coff).reshape(NC, NS, NBLK, CPB, K)
    edst = dst.reshape(NS, NBLK, CPB, K)
    mesh = plsc.VectorSubcoreMesh(core_axis_name="c", subcore_axis_name="s")
    fn = pl.kernel(
        _sc_body,
        out_type=jax.ShapeDtypeStruct((NC, NP, DH), jnp.float32),
        mesh=mesh,
        compiler_params=pltpu.CompilerParams(use_tc_tiling_on_sc=False),
        scratch_types=[
            pltpu.VMEM((2, CPB, K), jnp.int32),
            pltpu.VMEM((2, CPB, K), jnp.int32),
            pltpu.VMEM((K, DH), jnp.float32),
            pltpu.VMEM((K, DH), jnp.float32),
            pltpu.VMEM((K, DH), jnp.float32),
            pltpu.VMEM((K, DH), jnp.float32),
            pltpu.SemaphoreType.DMA,
            pltpu.SemaphoreType.DMA,
            pltpu.SemaphoreType.DMA,
            pltpu.SemaphoreType.DMA,
            pltpu.SemaphoreType.DMA,
            pltpu.SemaphoreType.DMA,
            pltpu.SemaphoreType.DMA,
            pltpu.SemaphoreType.DMA,
            pltpu.SemaphoreType.DMA,
            pltpu.VMEM_SHARED((NP, DH), jnp.float32),
        ],
    )
    return fn(esrc, edst, xs)


R = 2000
NB = N // R


def _tc_body(xb, ab, bb, W1b, b1b, W2b, b2b, W3b, b3b, outb):
    i = pl.program_id(0)
    h = xb[...] + jnp.concatenate([ab[0], ab[1]], axis=1)
    h = jnp.maximum(jnp.dot(h, W1b[...], preferred_element_type=jnp.float32)
                    + b1b[...], 0.0)
    h = jnp.maximum(jnp.dot(h, W2b[...], preferred_element_type=jnp.float32)
                    + b2b[...], 0.0)
    o = jnp.dot(h, W3b[...], preferred_element_type=jnp.float32) + b3b[...]
    gids = lax.broadcasted_iota(jnp.int32, (G, R), 0)
    onehot = (bb[0] == gids).astype(jnp.float32)
    seg = jnp.dot(onehot, o, preferred_element_type=jnp.float32)

    @pl.when(i == 0)
    def _():
        outb[...] = seg

    @pl.when(i > 0)
    def _():
        outb[...] += seg


@jax.jit
def _tc_mlp_pool(x, agg, batch, W1, b1, W2, b2, W3, b3):
    O = W3.shape[1]
    b3d = batch.reshape(NB, 1, R)
    full = lambda *_: (0, 0)
    out = pl.pallas_call(
        _tc_body,
        grid=(NB,),
        in_specs=[
            pl.BlockSpec((R, D), lambda i: (i, 0)),
            pl.BlockSpec((NC, R, DH), lambda i: (0, i, 0)),
            pl.BlockSpec((1, 1, R), lambda i: (i, 0, 0)),
            pl.BlockSpec((D, D), full),
            pl.BlockSpec((1, D), full),
            pl.BlockSpec((D, D), full),
            pl.BlockSpec((1, D), full),
            pl.BlockSpec((D, O), full),
            pl.BlockSpec((1, O), full),
        ],
        out_specs=pl.BlockSpec((G, O), full),
        out_shape=jax.ShapeDtypeStruct((G, O), jnp.float32),
    )(x, agg, b3d, W1, b1.reshape(1, D), W2, b2.reshape(1, D),
      W3, b3.reshape(1, O))
    return out


def kernel(x, edge_index, batch, W1, b1, W2, b2, W3, b3):
    agg = _sc_aggregate(edge_index, x)
    return _tc_mlp_pool(x, agg, batch, W1, b1, W2, b2, W3, b3)


# speedup vs baseline: 1.3014x; 1.3014x over previous
import functools

import jax
import jax.numpy as jnp
from jax import lax
from jax.experimental import pallas as pl
from jax.experimental.pallas import tpu as pltpu
from jax.experimental.pallas import tpu_sc as plsc

N = 10000
E = 320000
D = 128
G = 64

NC = 2
NS = 16
NW = NC * NS
EW = E // NW
K = 112
CPB = 6
NBLK = 15
EWP = NBLK * CPB * K
PADW = EWP - EW
NP = 10112
RPT = NP // NS
NPAD = NP - N


def _sc_body(er_hbm, x_hbm, out_hbm, sidx, didx, b0, b1, b2,
             g0, g1, g2, s0, s1, s2, isem, shared):
    c = lax.axis_index("c")
    s = lax.axis_index("s")
    w = c * NS + s
    bufs = (b0, b1, b2)
    gsems = (g0, g1, g2)
    ssems = (s0, s1, s2)

    zero = jnp.zeros((16,), jnp.float32)

    def _zfill(i, carry):
        b0[i // 8, pl.ds((i % 8) * 16, 16)] = zero
        return carry

    lax.fori_loop(0, K * 8, _zfill, 0)
    for z in range(5):
        pltpu.sync_copy(b0, shared.at[pl.ds(s * RPT + z * K, K)])
    pltpu.sync_copy(b0.at[pl.ds(0, RPT - 5 * K)],
                    shared.at[pl.ds(s * RPT + 5 * K, RPT - 5 * K)])

    pltpu.async_copy(x_hbm.at[pl.ds(0, K)], b1, s1)
    pltpu.async_copy(x_hbm.at[pl.ds(K, K)], b2, s2)

    plsc.subcore_barrier()

    def _gather(ph, j, q):
        pltpu.async_copy(x_hbm.at[sidx.at[ph, j]], bufs[q], gsems[q])

    def _gwait(ph, j, q):
        pltpu.make_async_copy(x_hbm.at[sidx.at[ph, j]], bufs[q],
                              gsems[q]).wait()

    def _scatter(ph, j, q):
        pltpu.async_copy(bufs[q], shared.at[didx.at[ph, j]], ssems[q],
                         add=True)

    def _swait(q):
        pltpu.make_async_copy(bufs[q], shared.at[didx.at[0, 0]],
                              ssems[q]).wait()

    pltpu.sync_copy(er_hbm.at[0, w, 0], sidx.at[0])
    pltpu.sync_copy(er_hbm.at[1, w, 0], didx.at[0])
    _gather(0, 0, 0)

    def _block(blk, carry):
        ph = lax.rem(blk, 2)
        for j in range(CPB):
            tgt = j + 1
            if tgt < CPB:
                _swait(tgt % 3)
                _gather(ph, tgt, tgt % 3)
            if j == 2:
                @pl.when(blk < NBLK - 1)
                def _():
                    pltpu.async_copy(er_hbm.at[0, w, blk + 1],
                                     sidx.at[1 - ph], isem)
                    pltpu.async_copy(er_hbm.at[1, w, blk + 1],
                                     didx.at[1 - ph], isem)
            if j == 4:
                @pl.when(blk < NBLK - 1)
                def _():
                    pltpu.make_async_copy(er_hbm.at[0, w, blk + 1],
                                          sidx.at[1 - ph], isem).wait()
                    pltpu.make_async_copy(er_hbm.at[1, w, blk + 1],
                                          didx.at[1 - ph], isem).wait()
            if j == CPB - 1:
                @pl.when(blk < NBLK - 1)
                def _():
                    _swait(0)
                    _gather(1 - ph, 0, 0)
            _gwait(ph, j, j % 3)
            _scatter(ph, j, j % 3)
        return carry

    lax.fori_loop(0, NBLK, _block, 0)
    for q in range(3):
        _swait(q)

    plsc.subcore_barrier()
    pltpu.sync_copy(shared.at[pl.ds(s * RPT, RPT)],
                    out_hbm.at[c, pl.ds(s * RPT, RPT)])


@jax.jit
def _sc_aggregate(edge_index, x):
    ei = edge_index.reshape(2, NW, EW)
    j = jnp.arange(PADW, dtype=jnp.int32)
    wv = jnp.arange(NW, dtype=jnp.int32)[:, None]
    pad_src = (wv * 317 + j * 13) % N
    pad_dst = (N + (wv * 31 + j) % NPAD).astype(jnp.int32)
    er = jnp.concatenate(
        [ei, jnp.stack([pad_src, pad_dst])], axis=2
    ).reshape(2, NW, NBLK, CPB, K)
    mesh = plsc.VectorSubcoreMesh(core_axis_name="c", subcore_axis_name="s")
    fn = pl.kernel(
        _sc_body,
        out_type=jax.ShapeDtypeStruct((NC, NP, D), jnp.float32),
        mesh=mesh,
        scratch_types=[
            pltpu.VMEM((2, CPB, K), jnp.int32),
            pltpu.VMEM((2, CPB, K), jnp.int32),
            pltpu.VMEM((K, D), jnp.float32),
            pltpu.VMEM((K, D), jnp.float32),
            pltpu.VMEM((K, D), jnp.float32),
            pltpu.SemaphoreType.DMA,
            pltpu.SemaphoreType.DMA,
            pltpu.SemaphoreType.DMA,
            pltpu.SemaphoreType.DMA,
            pltpu.SemaphoreType.DMA,
            pltpu.SemaphoreType.DMA,
            pltpu.SemaphoreType.DMA,
            pltpu.VMEM_SHARED((NP, D), jnp.float32),
        ],
    )
    return fn(er, x)


R = 2000
NB = N // R


def _tc_body(xb, ab, bb, W1b, b1b, W2b, b2b, W3b, b3b, outb):
    i = pl.program_id(0)
    h = xb[...] + ab[0] + ab[1]
    h = jnp.maximum(jnp.dot(h, W1b[...], preferred_element_type=jnp.float32)
                    + b1b[...], 0.0)
    h = jnp.maximum(jnp.dot(h, W2b[...], preferred_element_type=jnp.float32)
                    + b2b[...], 0.0)
    o = jnp.dot(h, W3b[...], preferred_element_type=jnp.float32) + b3b[...]
    gids = lax.broadcasted_iota(jnp.int32, (G, R), 0)
    onehot = (bb[0] == gids).astype(jnp.float32)
    seg = jnp.dot(onehot, o, preferred_element_type=jnp.float32)

    @pl.when(i == 0)
    def _():
        outb[...] = seg

    @pl.when(i > 0)
    def _():
        outb[...] += seg


@jax.jit
def _tc_mlp_pool(x, agg, batch, W1, b1, W2, b2, W3, b3):
    O = W3.shape[1]
    b3d = batch.reshape(NB, 1, R)
    full = lambda *_: (0, 0)
    out = pl.pallas_call(
        _tc_body,
        grid=(NB,),
        in_specs=[
            pl.BlockSpec((R, D), lambda i: (i, 0)),
            pl.BlockSpec((NC, R, D), lambda i: (0, i, 0)),
            pl.BlockSpec((1, 1, R), lambda i: (i, 0, 0)),
            pl.BlockSpec((D, D), full),
            pl.BlockSpec((1, D), full),
            pl.BlockSpec((D, D), full),
            pl.BlockSpec((1, D), full),
            pl.BlockSpec((D, O), full),
            pl.BlockSpec((1, O), full),
        ],
        out_specs=pl.BlockSpec((G, O), full),
        out_shape=jax.ShapeDtypeStruct((G, O), jnp.float32),
    )(x, agg, b3d, W1, b1.reshape(1, D), W2, b2.reshape(1, D),
      W3, b3.reshape(1, O))
    return out


def kernel(x, edge_index, batch, W1, b1, W2, b2, W3, b3):
    agg = _sc_aggregate(edge_index, x)
    return _tc_mlp_pool(x, agg, batch, W1, b1, W2, b2, W3, b3)


# speedup vs baseline: 1.3026x; 1.0009x over previous
import functools

import jax
import jax.numpy as jnp
from jax import lax
from jax.experimental import pallas as pl
from jax.experimental.pallas import tpu as pltpu
from jax.experimental.pallas import tpu_sc as plsc

N = 10000
E = 320000
D = 128
G = 64

NC = 2
NS = 16
NW = NC * NS
EW = E // NW
K = 84
CPB = 4
NBLK = 30
EWP = NBLK * CPB * K
PADW = EWP - EW
NP = 10112
RPT = NP // NS
NPAD = NP - N


def _sc_body(er_hbm, x_hbm, out_hbm, sidx, didx, b0, b1, b2, b3,
             g0, g1, g2, g3, s0, s1, s2, s3, isem, shared):
    c = lax.axis_index("c")
    s = lax.axis_index("s")
    w = c * NS + s
    bufs = (b0, b1, b2, b3)
    gsems = (g0, g1, g2, g3)
    ssems = (s0, s1, s2, s3)

    zero = jnp.zeros((16,), jnp.float32)

    def _zfill(i, carry):
        b0[i // 8, pl.ds((i % 8) * 16, 16)] = zero
        return carry

    lax.fori_loop(0, K * 8, _zfill, 0)
    for z in range(7):
        pltpu.sync_copy(b0.at[pl.ds(0, 80)],
                        shared.at[pl.ds(s * RPT + z * 80, 80)])
    pltpu.sync_copy(b0.at[pl.ds(0, RPT - 560)],
                    shared.at[pl.ds(s * RPT + 560, RPT - 560)])

    plsc.subcore_barrier()

    def _gather(ph, j, q):
        pltpu.async_copy(x_hbm.at[sidx.at[ph, j]], bufs[q], gsems[q])

    def _gwait(ph, j, q):
        pltpu.make_async_copy(x_hbm.at[sidx.at[ph, j]], bufs[q],
                              gsems[q]).wait()

    def _scatter(ph, j, q):
        pltpu.async_copy(bufs[q], shared.at[didx.at[ph, j]], ssems[q],
                         add=True)

    def _swait(q):
        pltpu.make_async_copy(bufs[q], shared.at[didx.at[0, 0]],
                              ssems[q]).wait()

    pltpu.sync_copy(er_hbm.at[0, w, 0], sidx.at[0])
    pltpu.sync_copy(er_hbm.at[1, w, 0], didx.at[0])
    pltpu.async_copy(er_hbm.at[0, w, 1], sidx.at[1], isem)
    pltpu.async_copy(er_hbm.at[1, w, 1], didx.at[1], isem)
    _gather(0, 0, 0)
    _gather(0, 1, 1)

    def _block(blk, carry):
        p3 = lax.rem(blk, 3)
        n3 = lax.rem(blk + 1, 3)
        for j in range(CPB):
            tgt = j + 2
            if tgt < CPB:
                @pl.when(blk > 0)
                def _():
                    _swait(tgt)
                _gather(p3, tgt, tgt)
            if j == 2:
                @pl.when(blk < NBLK - 1)
                def _():
                    pltpu.make_async_copy(er_hbm.at[0, w, blk + 1],
                                          sidx.at[n3], isem).wait()
                    pltpu.make_async_copy(er_hbm.at[1, w, blk + 1],
                                          didx.at[n3], isem).wait()

                @pl.when(blk < NBLK - 2)
                def _():
                    pltpu.async_copy(er_hbm.at[0, w, blk + 2],
                                     sidx.at[lax.rem(blk + 2, 3)], isem)
                    pltpu.async_copy(er_hbm.at[1, w, blk + 2],
                                     didx.at[lax.rem(blk + 2, 3)], isem)

                @pl.when(blk < NBLK - 1)
                def _():
                    _swait(0)
                    _gather(n3, 0, 0)
            if j == 3:
                @pl.when(blk < NBLK - 1)
                def _():
                    _swait(1)
                    _gather(n3, 1, 1)
            _gwait(p3, j, j)
            _scatter(p3, j, j)
        return carry

    lax.fori_loop(0, NBLK, _block, 0)
    for q in range(4):
        _swait(q)

    plsc.subcore_barrier()
    pltpu.sync_copy(shared.at[pl.ds(s * RPT, RPT)],
                    out_hbm.at[c, pl.ds(s * RPT, RPT)])


@jax.jit
def _sc_aggregate(edge_index, x):
    ei = edge_index.reshape(2, NW, EW)
    j = jnp.arange(PADW, dtype=jnp.int32)
    wv = jnp.arange(NW, dtype=jnp.int32)[:, None]
    pad_src = (wv * 317 + j * 13) % N
    pad_dst = (N + (wv * 31 + j) % NPAD).astype(jnp.int32)
    er = jnp.concatenate(
        [ei, jnp.stack([pad_src, pad_dst])], axis=2
    ).reshape(2, NW, NBLK, CPB, K)
    mesh = plsc.VectorSubcoreMesh(core_axis_name="c", subcore_axis_name="s")
    fn = pl.kernel(
        _sc_body,
        out_type=jax.ShapeDtypeStruct((NC, NP, D), jnp.float32),
        mesh=mesh,
        scratch_types=[
            pltpu.VMEM((3, CPB, K), jnp.int32),
            pltpu.VMEM((3, CPB, K), jnp.int32),
            pltpu.VMEM((K, D), jnp.float32),
            pltpu.VMEM((K, D), jnp.float32),
            pltpu.VMEM((K, D), jnp.float32),
            pltpu.VMEM((K, D), jnp.float32),
            pltpu.SemaphoreType.DMA,
            pltpu.SemaphoreType.DMA,
            pltpu.SemaphoreType.DMA,
            pltpu.SemaphoreType.DMA,
            pltpu.SemaphoreType.DMA,
            pltpu.SemaphoreType.DMA,
            pltpu.SemaphoreType.DMA,
            pltpu.SemaphoreType.DMA,
            pltpu.SemaphoreType.DMA,
            pltpu.VMEM_SHARED((NP, D), jnp.float32),
        ],
    )
    return fn(er, x)


R = 2000
NB = N // R


def _tc_body(xb, ab, bb, W1b, b1b, W2b, b2b, W3b, b3b, outb):
    i = pl.program_id(0)
    h = xb[...] + ab[0] + ab[1]
    h = jnp.maximum(jnp.dot(h, W1b[...], preferred_element_type=jnp.float32)
                    + b1b[...], 0.0)
    h = jnp.maximum(jnp.dot(h, W2b[...], preferred_element_type=jnp.float32)
                    + b2b[...], 0.0)
    o = jnp.dot(h, W3b[...], preferred_element_type=jnp.float32) + b3b[...]
    gids = lax.broadcasted_iota(jnp.int32, (G, R), 0)
    onehot = (bb[0] == gids).astype(jnp.float32)
    seg = jnp.dot(onehot, o, preferred_element_type=jnp.float32)

    @pl.when(i == 0)
    def _():
        outb[...] = seg

    @pl.when(i > 0)
    def _():
        outb[...] += seg


@jax.jit
def _tc_mlp_pool(x, agg, batch, W1, b1, W2, b2, W3, b3):
    O = W3.shape[1]
    b3d = batch.reshape(NB, 1, R)
    full = lambda *_: (0, 0)
    out = pl.pallas_call(
        _tc_body,
        grid=(NB,),
        in_specs=[
            pl.BlockSpec((R, D), lambda i: (i, 0)),
            pl.BlockSpec((NC, R, D), lambda i: (0, i, 0)),
            pl.BlockSpec((1, 1, R), lambda i: (i, 0, 0)),
            pl.BlockSpec((D, D), full),
            pl.BlockSpec((1, D), full),
            pl.BlockSpec((D, D), full),
            pl.BlockSpec((1, D), full),
            pl.BlockSpec((D, O), full),
            pl.BlockSpec((1, O), full),
        ],
        out_specs=pl.BlockSpec((G, O), full),
        out_shape=jax.ShapeDtypeStruct((G, O), jnp.float32),
    )(x, agg, b3d, W1, b1.reshape(1, D), W2, b2.reshape(1, D),
      W3, b3.reshape(1, O))
    return out


def kernel(x, edge_index, batch, W1, b1, W2, b2, W3, b3):
    agg = _sc_aggregate(edge_index, x)
    return _tc_mlp_pool(x, agg, batch, W1, b1, W2, b2, W3, b3)


# speedup vs baseline: 1.3148x; 1.0094x over previous
import functools

import jax
import jax.numpy as jnp
from jax import lax
from jax.experimental import pallas as pl
from jax.experimental.pallas import tpu as pltpu
from jax.experimental.pallas import tpu_sc as plsc

N = 10000
E = 320000
D = 128
G = 64

NC = 2
NS = 16
NW = NC * NS
EW = E // NW
K = 84
CPB = 4
NBLK = 30
EWP = NBLK * CPB * K
PADW = EWP - EW
NP = 10112
RPT = NP // NS
NPAD = NP - N


def _sc_body(er_hbm, x_hbm, out_hbm, sidx, didx, b0, b1, b2, b3,
             g0, g1, g2, g3, s0, s1, s2, s3, isem, shared):
    c = lax.axis_index("c")
    s = lax.axis_index("s")
    w = c * NS + s
    bufs = (b0, b1, b2, b3)
    gsems = (g0, g1, g2, g3)
    ssems = (s0, s1, s2, s3)

    def _gather(ph, j, q):
        pltpu.async_copy(x_hbm.at[sidx.at[ph, j]], bufs[q], gsems[q])

    def _gwait(ph, j, q):
        pltpu.make_async_copy(x_hbm.at[sidx.at[ph, j]], bufs[q],
                              gsems[q]).wait()

    def _scatter(ph, j, q):
        pltpu.async_copy(bufs[q], shared.at[didx.at[ph, j]], ssems[q],
                         add=True)

    def _swait(q):
        pltpu.make_async_copy(bufs[q], shared.at[didx.at[0, 0]],
                              ssems[q]).wait()

    pltpu.sync_copy(er_hbm.at[0, w, 0], sidx.at[0])
    pltpu.sync_copy(er_hbm.at[1, w, 0], didx.at[0])
    pltpu.async_copy(er_hbm.at[0, w, 1], sidx.at[1], isem)
    pltpu.async_copy(er_hbm.at[1, w, 1], didx.at[1], isem)
    _gather(0, 0, 0)
    _gather(0, 1, 1)

    zero = jnp.zeros((16,), jnp.float32)

    def _zfill(i, carry):
        b3[i // 8, pl.ds((i % 8) * 16, 16)] = zero
        return carry

    lax.fori_loop(0, 80 * 8, _zfill, 0)
    for z in range(7):
        pltpu.sync_copy(b3.at[pl.ds(0, 80)],
                        shared.at[pl.ds(s * RPT + z * 80, 80)])
    pltpu.sync_copy(b3.at[pl.ds(0, RPT - 560)],
                    shared.at[pl.ds(s * RPT + 560, RPT - 560)])

    plsc.subcore_barrier()

    def _block(blk, carry):
        p3 = lax.rem(blk, 3)
        n3 = lax.rem(blk + 1, 3)
        for j in range(CPB):
            tgt = j + 2
            if tgt < CPB:
                @pl.when(blk > 0)
                def _():
                    _swait(tgt)
                _gather(p3, tgt, tgt)
            if j == 2:
                @pl.when(blk < NBLK - 1)
                def _():
                    pltpu.make_async_copy(er_hbm.at[0, w, blk + 1],
                                          sidx.at[n3], isem).wait()
                    pltpu.make_async_copy(er_hbm.at[1, w, blk + 1],
                                          didx.at[n3], isem).wait()

                @pl.when(blk < NBLK - 2)
                def _():
                    pltpu.async_copy(er_hbm.at[0, w, blk + 2],
                                     sidx.at[lax.rem(blk + 2, 3)], isem)
                    pltpu.async_copy(er_hbm.at[1, w, blk + 2],
                                     didx.at[lax.rem(blk + 2, 3)], isem)

                @pl.when(blk < NBLK - 1)
                def _():
                    _swait(0)
                    _gather(n3, 0, 0)
            if j == 3:
                @pl.when(blk < NBLK - 1)
                def _():
                    _swait(1)
                    _gather(n3, 1, 1)
            _gwait(p3, j, j)
            _scatter(p3, j, j)
        return carry

    lax.fori_loop(0, NBLK, _block, 0)
    for q in range(4):
        _swait(q)

    plsc.subcore_barrier()
    pltpu.sync_copy(shared.at[pl.ds(s * RPT, RPT)],
                    out_hbm.at[c, pl.ds(s * RPT, RPT)])


@jax.jit
def _sc_aggregate(edge_index, x):
    ei = edge_index.reshape(2, NW, EW)
    j = jnp.arange(PADW, dtype=jnp.int32)
    wv = jnp.arange(NW, dtype=jnp.int32)[:, None]
    pad_src = (wv * 317 + j * 13) % N
    pad_dst = (N + (wv * 31 + j) % NPAD).astype(jnp.int32)
    er = jnp.concatenate(
        [ei, jnp.stack([pad_src, pad_dst])], axis=2
    ).reshape(2, NW, NBLK, CPB, K)
    mesh = plsc.VectorSubcoreMesh(core_axis_name="c", subcore_axis_name="s")
    fn = pl.kernel(
        _sc_body,
        out_type=jax.ShapeDtypeStruct((NC, NP, D), jnp.float32),
        mesh=mesh,
        scratch_types=[
            pltpu.VMEM((3, CPB, K), jnp.int32),
            pltpu.VMEM((3, CPB, K), jnp.int32),
            pltpu.VMEM((K, D), jnp.float32),
            pltpu.VMEM((K, D), jnp.float32),
            pltpu.VMEM((K, D), jnp.float32),
            pltpu.VMEM((K, D), jnp.float32),
            pltpu.SemaphoreType.DMA,
            pltpu.SemaphoreType.DMA,
            pltpu.SemaphoreType.DMA,
            pltpu.SemaphoreType.DMA,
            pltpu.SemaphoreType.DMA,
            pltpu.SemaphoreType.DMA,
            pltpu.SemaphoreType.DMA,
            pltpu.SemaphoreType.DMA,
            pltpu.SemaphoreType.DMA,
            pltpu.VMEM_SHARED((NP, D), jnp.float32),
        ],
    )
    return fn(er, x)


R = 2000
NB = N // R


def _tc_body(xb, ab, bb, W1b, b1b, W2b, b2b, W3b, b3b, outb):
    i = pl.program_id(0)
    h = xb[...] + ab[0] + ab[1]
    h = jnp.maximum(jnp.dot(h, W1b[...], preferred_element_type=jnp.float32)
                    + b1b[...], 0.0)
    h = jnp.maximum(jnp.dot(h, W2b[...], preferred_element_type=jnp.float32)
                    + b2b[...], 0.0)
    o = jnp.dot(h, W3b[...], preferred_element_type=jnp.float32) + b3b[...]
    gids = lax.broadcasted_iota(jnp.int32, (G, R), 0)
    onehot = (bb[0] == gids).astype(jnp.float32)
    seg = jnp.dot(onehot, o, preferred_element_type=jnp.float32)

    @pl.when(i == 0)
    def _():
        outb[...] = seg

    @pl.when(i > 0)
    def _():
        outb[...] += seg


@jax.jit
def _tc_mlp_pool(x, agg, batch, W1, b1, W2, b2, W3, b3):
    O = W3.shape[1]
    b3d = batch.reshape(NB, 1, R)
    full = lambda *_: (0, 0)
    out = pl.pallas_call(
        _tc_body,
        grid=(NB,),
        in_specs=[
            pl.BlockSpec((R, D), lambda i: (i, 0)),
            pl.BlockSpec((NC, R, D), lambda i: (0, i, 0)),
            pl.BlockSpec((1, 1, R), lambda i: (i, 0, 0)),
            pl.BlockSpec((D, D), full),
            pl.BlockSpec((1, D), full),
            pl.BlockSpec((D, D), full),
            pl.BlockSpec((1, D), full),
            pl.BlockSpec((D, O), full),
            pl.BlockSpec((1, O), full),
        ],
        out_specs=pl.BlockSpec((G, O), full),
        out_shape=jax.ShapeDtypeStruct((G, O), jnp.float32),
    )(x, agg, b3d, W1, b1.reshape(1, D), W2, b2.reshape(1, D),
      W3, b3.reshape(1, O))
    return out


def kernel(x, edge_index, batch, W1, b1, W2, b2, W3, b3):
    agg = _sc_aggregate(edge_index, x)
    return _tc_mlp_pool(x, agg, batch, W1, b1, W2, b2, W3, b3)
